# Initial kernel scaffold; baseline (speedup 1.0000x reference)
import jax
import jax.numpy as jnp
from jax.experimental import pallas as pl


def kernel(x):
    raise NotImplementedError("write your pallas kernel here")


# speedup vs baseline: 6.7856x; 6.7856x over previous
import jax
import jax.numpy as jnp
from jax.experimental import pallas as pl

OUT = 128
MASKING = -1000000000.0
BM = 2048

COL_BACK = 256 - 10 - 128
COL_FWD = 256 - 6 - 128
COL_LEFT = 256 - 5 - 128
COL_RIGHT = 256 - 1 - 128


def _mask_kernel(x_ref, o_ref):
    back = x_ref[:, COL_BACK:COL_BACK + 1]
    fwd = x_ref[:, COL_FWD:COL_FWD + 1]
    left = x_ref[:, COL_LEFT:COL_LEFT + 1]
    right = x_ref[:, COL_RIGHT:COL_RIGHT + 1]
    col = jax.lax.broadcasted_iota(jnp.int32, (BM, OUT), 1)
    hit = ((col == 1) & (fwd == 1.0)) | ((col == 2) & (back == 1.0)) \
        | ((col == 3) & (left == 1.0)) | ((col == 4) & (right == 1.0))
    o_ref[...] = jnp.where(hit, MASKING, 1.0)


def kernel(x):
    B = x.shape[0]
    return pl.pallas_call(
        _mask_kernel,
        grid=(B // BM,),
        in_specs=[pl.BlockSpec((BM, 128), lambda i: (i, 1))],
        out_specs=pl.BlockSpec((BM, OUT), lambda i: (i, 0)),
        out_shape=jax.ShapeDtypeStruct((B, OUT), jnp.float32),
    )(x)
